# initial kernel scaffold (unmeasured)
import jax
import jax.numpy as jnp
from jax import lax
from jax.experimental import pallas as pl
from jax.experimental.pallas import tpu as pltpu


def kernel(
    x,
):
    def body(*refs):
        pass

    out_shape = jax.ShapeDtypeStruct(..., jnp.float32)
    return pl.pallas_call(body, out_shape=out_shape)(...)


# baseline (device time: 176567 ns/iter reference)
import jax
import jax.numpy as jnp
from jax import lax
from jax.experimental import pallas as pl
from jax.experimental.pallas import tpu as pltpu

N_DEV = 16
LOG_DEV = 4
N_EXCH = LOG_DEV * (LOG_DEV + 1) // 2


def kernel(x):
    m, n = x.shape
    log_m = m.bit_length() - 1
    log_total = log_m + LOG_DEV

    def ce_pass(v, d, s, my):
        up = jnp.concatenate([v[d:], v[:d]], axis=0)
        down = jnp.concatenate([v[-d:], v[:-d]], axis=0)
        r_idx = lax.broadcasted_iota(jnp.int32, (m, 1), 0)
        is_high = (r_idx & d) != 0
        partner = jnp.where(is_high, down, up)
        mn = jnp.minimum(v, partner)
        mx = jnp.maximum(v, partner)
        asc = (((my * m + r_idx) >> s) & 1) == 0
        keep_min = asc != is_high
        return jnp.where(keep_min, mn, mx)

    def body(x_ref, o_ref, cur_ref, recv_ref, send_sems, recv_sems):
        my = lax.axis_index("i")

        barrier_sem = pltpu.get_barrier_semaphore()
        for dd in (1, 2, 4, 8):
            pl.semaphore_signal(
                barrier_sem,
                inc=1,
                device_id=(my ^ dd,),
                device_id_type=pl.DeviceIdType.MESH,
            )
        pl.semaphore_wait(barrier_sem, LOG_DEV)

        v = x_ref[:, :]
        for s in range(1, log_m + 1):
            for j in range(s - 1, -1, -1):
                v = ce_pass(v, 2 ** j, s, my)
        cur_ref[:, :] = v

        k = 0
        for s in range(log_m + 1, log_total + 1):
            for j in range(s - 1 - log_m, -1, -1):
                d_dev = 2 ** j
                partner = my ^ d_dev
                rdma = pltpu.make_async_remote_copy(
                    src_ref=cur_ref,
                    dst_ref=recv_ref.at[k],
                    send_sem=send_sems.at[k],
                    recv_sem=recv_sems.at[k],
                    device_id=(partner,),
                    device_id_type=pl.DeviceIdType.MESH,
                )
                rdma.start()
                rdma.wait()

                i_am_lo = (my & d_dev) == 0
                asc = ((my >> (s - log_m)) & 1) == 0
                keep_min = i_am_lo == asc
                c = cur_ref[:, :]
                r = recv_ref[k, :, :]
                v = jnp.where(
                    keep_min, jnp.minimum(c, r), jnp.maximum(c, r)
                )
                cur_ref[:, :] = v
                k += 1

            for j in range(log_m - 1, -1, -1):
                v = ce_pass(v, 2 ** j, s, my)
            cur_ref[:, :] = v

        o_ref[:, :] = v

    return pl.pallas_call(
        body,
        out_shape=jax.ShapeDtypeStruct((m, n), x.dtype),
        in_specs=[pl.BlockSpec(memory_space=pltpu.VMEM)],
        out_specs=pl.BlockSpec(memory_space=pltpu.VMEM),
        scratch_shapes=[
            pltpu.VMEM((m, n), x.dtype),
            pltpu.VMEM((N_EXCH, m, n), x.dtype),
            pltpu.SemaphoreType.DMA((N_EXCH,)),
            pltpu.SemaphoreType.DMA((N_EXCH,)),
        ],
        compiler_params=pltpu.CompilerParams(collective_id=0),
    )(x)
